# Initial kernel scaffold; baseline (speedup 1.0000x reference)
import jax
import jax.numpy as jnp
from jax.experimental import pallas as pl


def kernel(node_feature, edge_index, edge_feature, W):
    raise NotImplementedError("write your pallas kernel here")


# speedup vs baseline: 3.0686x; 3.0686x over previous
import jax
import jax.numpy as jnp
from jax import lax
from jax.experimental import pallas as pl
from jax.experimental.pallas import tpu as pltpu
from jax.experimental.pallas import tpu_sc as plsc

NC = 2
NS = 16

CH = 128
N_NODES = 10000
N_ACC = 10240
D_IN = 128
D_HALF = D_IN // NC
D_EDGE = 16
E_EDGES = 320000
N_CHUNKS = E_EDGES // CH


def _sc_body(node_hbm, src_hbm, dst_hbm, ef_hbm, a_out, b_out,
             a_acc, b_acc, src_v, dst_v, rows_v, ef_v, sem):
    cid = lax.axis_index("c")
    sid = lax.axis_index("s")

    zv = jnp.zeros((16,), jnp.float32)
    cpr = D_HALF // 16

    def _zrow(i, c):
        rows_v[i // cpr, pl.ds((i % cpr) * 16, 16)] = zv
        return c

    lax.fori_loop(0, (CH * D_HALF) // 16, _zrow, 0)

    def _zef(i, c):
        ef_v[i, :] = zv
        return c

    lax.fori_loop(0, CH, _zef, 0)

    for z in range(N_ACC // NS // CH):
        base = sid * (N_ACC // NS) + z * CH
        pltpu.sync_copy(rows_v, a_acc.at[pl.ds(base, CH)])
        pltpu.sync_copy(ef_v, b_acc.at[pl.ds(base, CH)])

    plsc.subcore_barrier()

    n_ch = jnp.where(sid < (N_CHUNKS % NS), N_CHUNKS // NS + 1, N_CHUNKS // NS)

    def _chunk(ch, c):
        g = sid + ch * NS
        pltpu.sync_copy(src_hbm.at[g], src_v)
        pltpu.sync_copy(dst_hbm.at[g], dst_v)
        pltpu.sync_copy(ef_hbm.at[g], ef_v)
        pltpu.async_copy(node_hbm.at[cid].at[src_v.at[0]], rows_v, sem).wait()
        pltpu.sync_copy(rows_v, a_acc.at[dst_v.at[0]], add=True)
        pltpu.sync_copy(ef_v, b_acc.at[dst_v.at[0]], add=True)
        return c

    lax.fori_loop(0, n_ch, _chunk, 0)

    plsc.subcore_barrier()

    out_rows = N_ACC // NS
    obase = sid * out_rows
    pltpu.sync_copy(a_acc.at[pl.ds(obase, out_rows)],
                    a_out.at[cid, pl.ds(obase, out_rows)])
    pltpu.sync_copy(b_acc.at[pl.ds(obase, out_rows)],
                    b_out.at[cid, pl.ds(obase, out_rows)])


_sc_call = pl.kernel(
    _sc_body,
    out_type=(
        jax.ShapeDtypeStruct((NC, N_ACC, D_HALF), jnp.float32),
        jax.ShapeDtypeStruct((NC, N_ACC, D_EDGE), jnp.float32),
    ),
    mesh=plsc.VectorSubcoreMesh(
        core_axis_name="c", subcore_axis_name="s",
        num_cores=NC, num_subcores=NS),
    compiler_params=pltpu.CompilerParams(use_tc_tiling_on_sc=False),
    scratch_types=[
        pltpu.VMEM_SHARED((N_ACC, D_HALF), jnp.float32),
        pltpu.VMEM_SHARED((N_ACC, D_EDGE), jnp.float32),
        pltpu.VMEM((1, CH), jnp.int32),
        pltpu.VMEM((1, CH), jnp.int32),
        pltpu.VMEM((CH, D_HALF), jnp.float32),
        pltpu.VMEM((CH, D_EDGE), jnp.float32),
        pltpu.SemaphoreType.DMA,
    ],
)


def _mm_body(a_ref, b_ref, w1_ref, w2_ref, o_ref):
    a = jnp.concatenate([a_ref[0], a_ref[1]], axis=-1)
    o_ref[...] = (
        jnp.dot(a, w1_ref[...], preferred_element_type=jnp.float32)
        + jnp.dot(b_ref[0], w2_ref[...], preferred_element_type=jnp.float32))


def _combine(A, B, W1, W2):
    blk = 1000
    grid = (N_NODES // blk,)
    return pl.pallas_call(
        _mm_body,
        grid=grid,
        in_specs=[
            pl.BlockSpec((NC, blk, D_HALF), lambda i: (0, i, 0)),
            pl.BlockSpec((1, blk, D_EDGE), lambda i: (0, i, 0)),
            pl.BlockSpec((D_IN, D_IN), lambda i: (0, 0)),
            pl.BlockSpec((D_EDGE, D_IN), lambda i: (0, 0)),
        ],
        out_specs=pl.BlockSpec((blk, D_IN), lambda i: (i, 0)),
        out_shape=jax.ShapeDtypeStruct((N_NODES, D_IN), jnp.float32),
    )(A, B, W1, W2)


def kernel(node_feature, edge_index, edge_feature, W):
    N, D = node_feature.shape
    E = edge_index.shape[1]
    assert (N, D, E) == (N_NODES, D_IN, E_EDGES)

    keep = jax.random.uniform(jax.random.key(42), (E,)) < 0.5
    src = edge_index[0]
    dst = jnp.where(keep, edge_index[1], N).astype(jnp.int32)

    node_half = node_feature.reshape(N, NC, D_HALF).transpose(1, 0, 2)
    src3 = src.reshape(N_CHUNKS, 1, CH)
    dst3 = dst.reshape(N_CHUNKS, 1, CH)
    ef3 = edge_feature.reshape(N_CHUNKS, CH, D_EDGE)

    A, B = _sc_call(node_half, src3, dst3, ef3)
    return _combine(A, B, W[:D], W[D:])
